# Initial kernel scaffold; baseline (speedup 1.0000x reference)
import jax
import jax.numpy as jnp
from jax.experimental import pallas as pl


def kernel(x):
    raise NotImplementedError("write your pallas kernel here")


# speedup vs baseline: 1.0427x; 1.0427x over previous
import functools

import jax
import jax.numpy as jnp
from jax.experimental import pallas as pl

_K = 20
_VLAD_K = 50
_N_ITER = 10


def _labels_body(f_ref, cb_ref, f2_ref, c2_ref, lab_ref):
    f = f_ref[...]
    cb = cb_ref[...]
    mm = jax.lax.dot_general(
        f, cb, (((1,), (1,)), ((), ())),
        preferred_element_type=jnp.float32)
    d = (f2_ref[...] - 2.0 * mm) + c2_ref[...]
    lab_ref[...] = jnp.argmin(d, axis=1, keepdims=True).astype(jnp.int32)


def _pallas_assign(feature, cents, f2):
    n, dd = feature.shape
    cb = cents.astype(jnp.bfloat16)
    c2 = jnp.sum(cents * cents, axis=1)
    labels = pl.pallas_call(
        _labels_body,
        out_shape=jax.ShapeDtypeStruct((n, 1), jnp.int32),
    )(feature, cb, f2[:, None], c2[None, :])
    return labels[:, 0]


def _vlad_one(y_i):
    n = y_i.shape[0]
    f2 = jnp.sum(y_i * y_i, axis=1)
    cents0 = y_i[:_K]

    def step(cents, _):
        labels = _pallas_assign(y_i, cents, f2)
        sums = jax.ops.segment_sum(y_i, labels, num_segments=_K)
        counts = jax.ops.segment_sum(jnp.ones((n,), y_i.dtype), labels,
                                     num_segments=_K)
        new = jnp.where(counts[:, None] > 0,
                        sums / jnp.maximum(counts, 1.0)[:, None],
                        cents)
        return new, None

    cents, _ = jax.lax.scan(step, cents0, None, length=_N_ITER)
    labels = _pallas_assign(y_i, cents, f2)
    resid = y_i - cents[labels]
    out = jnp.zeros((_VLAD_K, y_i.shape[1]), y_i.dtype).at[labels].add(resid)
    return out


def kernel(x):
    b, c, h, w = x.shape
    y = x.reshape(b, c, h * w)
    return jax.vmap(_vlad_one)(y)


# speedup vs baseline: 1.4745x; 1.4141x over previous
import functools

import jax
import jax.numpy as jnp
from jax.experimental import pallas as pl

_K = 20
_VLAD_K = 50
_N_ITER = 10


def _labels_body(f_ref, cb_ref, f2_ref, c2_ref, lab_ref, cnt_ref):
    f = f_ref[...]
    cb = cb_ref[...]
    n = f.shape[0]
    mm = jax.lax.dot_general(
        f, cb, (((1,), (1,)), ((), ())),
        preferred_element_type=jnp.float32)
    d = (f2_ref[...] - 2.0 * mm) + c2_ref[...]
    lab = jnp.argmin(d, axis=1, keepdims=True).astype(jnp.int32)
    lab_ref[...] = lab
    k_iota = jax.lax.broadcasted_iota(jnp.int32, (n, _K), 1)
    onehot = (lab == k_iota).astype(jnp.float32)
    cnt_ref[...] = jnp.sum(onehot, axis=0, keepdims=True)


def _pallas_assign(feature, cents, f2):
    n, dd = feature.shape
    cb = cents.astype(jnp.bfloat16)
    c2 = jnp.sum(cents * cents, axis=1)
    labels, counts = pl.pallas_call(
        _labels_body,
        out_shape=(jax.ShapeDtypeStruct((n, 1), jnp.int32),
                   jax.ShapeDtypeStruct((1, _K), jnp.float32)),
    )(feature, cb, f2[:, None], c2[None, :])
    return labels[:, 0], counts[0]


def _vlad_one(y_i):
    n = y_i.shape[0]
    f2 = jnp.sum(y_i * y_i, axis=1)
    cents0 = y_i[:_K]

    def step(cents, _):
        labels, counts = _pallas_assign(y_i, cents, f2)
        sums = jax.ops.segment_sum(y_i, labels, num_segments=_K)
        new = jnp.where(counts[:, None] > 0,
                        sums / jnp.maximum(counts, 1.0)[:, None],
                        cents)
        return new, None

    cents, _ = jax.lax.scan(step, cents0, None, length=_N_ITER)
    labels, _ = _pallas_assign(y_i, cents, f2)
    resid = y_i - cents[labels]
    out = jnp.zeros((_VLAD_K, y_i.shape[1]), y_i.dtype).at[labels].add(resid)
    return out


def kernel(x):
    b, c, h, w = x.shape
    y = x.reshape(b, c, h * w)
    return jax.vmap(_vlad_one)(y)
